# Initial kernel scaffold; baseline (speedup 1.0000x reference)
import jax
import jax.numpy as jnp
from jax.experimental import pallas as pl


def kernel(x, edge_index, batch, params):
    raise NotImplementedError("write your pallas kernel here")


# speedup vs baseline: 1.7428x; 1.7428x over previous
import functools
import math

import jax
import jax.numpy as jnp
import numpy as np
from jax.experimental import pallas as pl
from jax.experimental.pallas import tpu as pltpu

_N = 10000
_G = 64
_RATIO = 0.8
_H = 128
_BN_INV = np.float32(1.0 / np.sqrt(1.0 + 1e-5))

_ROW_BLK = 1000


def _proj_body(h_ref, wl_ref, wr_ref, xl_ref, xr_ref):
    h = h_ref[...]
    xl_ref[...] = jnp.dot(h, wl_ref[...], preferred_element_type=jnp.float32)
    xr_ref[...] = jnp.dot(h, wr_ref[...], preferred_element_type=jnp.float32)


def _proj(h, Wl, Wr):
    n = h.shape[0]
    grid = n // _ROW_BLK
    return pl.pallas_call(
        _proj_body,
        grid=(grid,),
        in_specs=[
            pl.BlockSpec((_ROW_BLK, _H), lambda i: (i, 0)),
            pl.BlockSpec((_H, _H), lambda i: (0, 0)),
            pl.BlockSpec((_H, _H), lambda i: (0, 0)),
        ],
        out_specs=[
            pl.BlockSpec((_ROW_BLK, _H), lambda i: (i, 0)),
            pl.BlockSpec((_ROW_BLK, _H), lambda i: (i, 0)),
        ],
        out_shape=[jax.ShapeDtypeStruct((n, _H), jnp.float32)] * 2,
    )(h, Wl, Wr)


def _combine_body(agg_ref, den_ref, xl_ref, xr_ref, att_ref, bias_ref,
                  g_ref, b_ref, wn_ref, h_ref, score_ref):
    xl = xl_ref[...]
    xr = xr_ref[...]
    m = xl + xr
    m = jnp.where(m >= 0, m, 0.2 * m)
    e0 = jnp.dot(m, att_ref[...], preferred_element_type=jnp.float32)
    ee0 = jnp.exp(e0)
    out = agg_ref[...] + ee0 * xl
    den = den_ref[...] + ee0
    h = out / den + bias_ref[...]
    h = g_ref[...] * (h * _BN_INV) + b_ref[...]
    h = jnp.maximum(h, 0.0)
    h_ref[...] = h
    score_ref[...] = jnp.tanh(
        jnp.dot(h, wn_ref[...], preferred_element_type=jnp.float32))


def _combine(agg, den, xl, xr, att, bias, g, b, wn):
    n = xl.shape[0]
    grid = n // _ROW_BLK
    vec = lambda i: (i, 0)
    fixed = lambda i: (0, 0)
    return pl.pallas_call(
        _combine_body,
        grid=(grid,),
        in_specs=[
            pl.BlockSpec((_ROW_BLK, _H), vec),
            pl.BlockSpec((_ROW_BLK, 1), vec),
            pl.BlockSpec((_ROW_BLK, _H), vec),
            pl.BlockSpec((_ROW_BLK, _H), vec),
            pl.BlockSpec((_H, 1), fixed),
            pl.BlockSpec((1, _H), fixed),
            pl.BlockSpec((1, _H), fixed),
            pl.BlockSpec((1, _H), fixed),
            pl.BlockSpec((_H, 1), fixed),
        ],
        out_specs=[
            pl.BlockSpec((_ROW_BLK, _H), vec),
            pl.BlockSpec((_ROW_BLK, 1), vec),
        ],
        out_shape=[
            jax.ShapeDtypeStruct((n, _H), jnp.float32),
            jax.ShapeDtypeStruct((n, 1), jnp.float32),
        ],
    )(agg, den, xl, xr, att, bias, g, b, wn)


def _mlp_body(f_ref, w1_ref, b1_ref, w2_ref, b2_ref, o_ref):
    hd = jnp.dot(f_ref[...], w1_ref[...], preferred_element_type=jnp.float32)
    hd = jnp.maximum(hd + b1_ref[...], 0.0)
    o_ref[...] = jnp.dot(hd, w2_ref[...],
                         preferred_element_type=jnp.float32) + b2_ref[...]


def _mlp(flat, W1, b1, W2, b2):
    g, fin = flat.shape
    hid = W1.shape[1]
    c = W2.shape[1]
    return pl.pallas_call(
        _mlp_body,
        in_specs=[pl.BlockSpec(flat.shape, lambda: (0, 0)),
                  pl.BlockSpec(W1.shape, lambda: (0, 0)),
                  pl.BlockSpec((1, hid), lambda: (0, 0)),
                  pl.BlockSpec(W2.shape, lambda: (0, 0)),
                  pl.BlockSpec((1, c), lambda: (0, 0))],
        out_specs=pl.BlockSpec((g, c), lambda: (0, 0)),
        out_shape=jax.ShapeDtypeStruct((g, c), jnp.float32),
    )(flat, W1, b1.reshape(1, hid), W2, b2.reshape(1, c))


def _edge_aggregate(xl, xr, att, src, dst, evalid):
    n = xl.shape[0]
    s = jnp.where(evalid, src, 0)
    d = jnp.where(evalid, dst, 0)
    seg = jnp.where(evalid, dst, n)
    xls = xl[s]
    m = xls + xr[d]
    m = jnp.where(m >= 0, m, 0.2 * m)
    e = m @ att
    ee = jnp.where(evalid, jnp.exp(e), 0.0)
    den = jax.ops.segment_sum(ee, seg, num_segments=n + 1)[:n]
    agg = jax.ops.segment_sum(xls * ee[:, None], seg, num_segments=n + 1)[:n]
    return agg, den


def _topk(score, batch, ei, valid, evalid):
    n = score.shape[0]
    bg = jnp.where(valid, batch, _G).astype(jnp.int32)
    order = jnp.lexsort((-score, bg))
    bs = bg[order]
    sizes = jax.ops.segment_sum(jnp.ones((n,), jnp.int32), bg,
                                num_segments=_G + 1)
    k = jnp.ceil(_RATIO * sizes.astype(jnp.float32)).astype(jnp.int32)
    starts = jnp.concatenate([jnp.zeros((1,), jnp.int32),
                              jnp.cumsum(sizes)[:-1].astype(jnp.int32)])
    rank = jnp.arange(n, dtype=jnp.int32) - starts[bs]
    keep = (rank < k[bs]) & (bs < _G)
    pos = jnp.cumsum(keep.astype(jnp.int32)) - 1
    mkeep = jnp.sum(keep.astype(jnp.int32))
    dest = jnp.where(keep, pos, n)
    newid = jnp.full((n,), -1, jnp.int32).at[order].set(
        jnp.where(keep, pos, -1).astype(jnp.int32))
    b_new = jnp.full((n + 1,), _G, jnp.int32).at[dest].set(bs)[:n]
    nsrc = newid[ei[0]]
    ndst = newid[ei[1]]
    ev_new = evalid & (nsrc >= 0) & (ndst >= 0)
    ei_new = jnp.stack([jnp.where(ev_new, nsrc, 0),
                        jnp.where(ev_new, ndst, 0)]).astype(jnp.int32)
    valid_new = jnp.arange(n, dtype=jnp.int32) < mkeep
    perm = jnp.zeros((n + 1,), jnp.int32).at[dest].set(order.astype(jnp.int32))[:n]
    return perm, mkeep, ei_new, b_new, valid_new, ev_new


def _gpool(h, b):
    add = jax.ops.segment_sum(h, b, num_segments=_G + 1)[:_G]
    mx = jax.ops.segment_max(h, b, num_segments=_G + 1)[:_G]
    mx = jnp.where(jnp.isfinite(mx), mx, 0.0)
    return jnp.concatenate([add, mx], axis=-1)


def kernel(x, edge_index, batch, params):
    n = x.shape[0]
    ei = edge_index.astype(jnp.int32)
    b = batch.astype(jnp.int32)
    valid = jnp.ones((n,), bool)
    evalid = jnp.ones((ei.shape[1],), bool)
    h = x
    flats = []
    for i in range(1, 5):
        p = params['conv%d' % i]
        xl, xr = _proj(h, p['Wl'], p['Wr'])
        agg, den = _edge_aggregate(xl, xr, p['att'], ei[0], ei[1], evalid)
        w = params['pool%d_w' % i]
        wn = w / jnp.sqrt(jnp.sum(w * w))
        h, score = _combine(agg, den[:, None], xl, xr,
                            p['att'].reshape(_H, 1),
                            p['bias'].reshape(1, _H),
                            params['bn%d_g' % i].reshape(1, _H),
                            params['bn%d_b' % i].reshape(1, _H),
                            wn.reshape(_H, 1))
        score = score[:, 0]
        perm, mkeep, ei, b, valid, evalid = _topk(score, b, ei, valid, evalid)
        hp = h[perm] * score[perm][:, None]
        h = jnp.where((jnp.arange(n) < mkeep)[:, None], hp, 0.0)
        flats.append(_gpool(h, b))
    flat = jnp.concatenate(flats, axis=-1)
    return _mlp(flat, params['W1'], params['b1'], params['W2'], params['b2'])
